# baseline (speedup 1.0000x reference)
import functools

import jax
import jax.numpy as jnp
from jax import lax
from jax.experimental import pallas as pl
from jax.experimental.pallas import tpu as pltpu
from jax.experimental.pallas import tpu_sc as plsc

EPS = 1e-12

NC = 2
NS = 16
NW = NC * NS

CH = 64
HP = 384
LANES = 16


def _ln_table_body(tbl_ref, g_ref, b_ref, out_ref):
    x = tbl_ref[...]
    u = jnp.mean(x, axis=-1, keepdims=True)
    s = jnp.mean((x - u) ** 2, axis=-1, keepdims=True)
    y = g_ref[...] * ((x - u) / jnp.sqrt(s + EPS)) + b_ref[...]
    out_ref[...] = jnp.pad(y, ((0, 0), (0, HP - y.shape[1])))


def _normalize_table(word_embeddings, gamma, beta):
    V, H = word_embeddings.shape
    BR = 2000
    grid = V // BR
    g2 = gamma.reshape(1, H)
    b2 = beta.reshape(1, H)
    return pl.pallas_call(
        _ln_table_body,
        grid=(grid,),
        in_specs=[
            pl.BlockSpec((BR, H), lambda i: (i, 0)),
            pl.BlockSpec((1, H), lambda i: (0, 0)),
            pl.BlockSpec((1, H), lambda i: (0, 0)),
        ],
        out_specs=pl.BlockSpec((BR, HP), lambda i: (i, 0)),
        out_shape=jax.ShapeDtypeStruct((V, HP), jnp.float32),
    )(word_embeddings, g2, b2)


def _make_sc_gather(ntok, H, n_chunks):
    mesh = plsc.VectorSubcoreMesh(core_axis_name="c", subcore_axis_name="s")
    per_w = n_chunks * CH
    nfull = H // LANES
    tail = H - LANES

    @functools.partial(
        pl.kernel,
        mesh=mesh,
        out_type=jax.ShapeDtypeStruct((ntok, H), jnp.float32),
        scratch_types=[
            pltpu.VMEM((n_chunks, CH), jnp.int32),
            pltpu.VMEM((2, CH, HP), jnp.float32),
            pltpu.VMEM((2, CH, H), jnp.float32),
            pltpu.SemaphoreType.DMA((2,)),
            pltpu.SemaphoreType.DMA((2,)),
        ],
    )
    def gather_kernel(tbl_hbm, idx_hbm, out_hbm, idx_v, r384, r300, gsem, ssem):
        wid = lax.axis_index("s") * NC + lax.axis_index("c")
        base = wid * per_w
        pltpu.sync_copy(idx_hbm.at[wid], idx_v)

        def gather_chunk(c, b):
            return pltpu.make_async_copy(
                tbl_hbm.at[idx_v.at[c]], r384.at[b], gsem.at[b])

        def store_chunk(c, b):
            row0 = pl.multiple_of(base + c * CH, 8)
            return pltpu.make_async_copy(
                r300.at[b], out_hbm.at[pl.ds(row0, CH)], ssem.at[b])

        gather_chunk(0, 0).start()
        gather_chunk(1, 1).start()

        def step(c0):
            for b in range(2):
                c = c0 + b
                gather_chunk(c, b).wait()

                @pl.when(c >= 2)
                def _():
                    store_chunk(c, b).wait()

                def repack(j):
                    for k in range(nfull):
                        r300[b, j, pl.ds(LANES * k, LANES)] = (
                            r384[b, j, pl.ds(LANES * k, LANES)])
                    r300[b, j, pl.ds(tail, LANES)] = r384[b, j, pl.ds(tail, LANES)]

                pl.loop(0, CH)(repack)
                store_chunk(c, b).start()

                @pl.when(c + 2 < n_chunks)
                def _():
                    gather_chunk(c + 2, b).start()

        pl.loop(0, n_chunks, step=2)(step)
        store_chunk(n_chunks - 2, 0).wait()
        store_chunk(n_chunks - 1, 1).wait()

    return gather_kernel


def kernel(input_ids, word_embeddings, gamma, beta):
    B, L = input_ids.shape
    V, H = word_embeddings.shape
    ntok = B * L
    n_chunks = ntok // (NW * CH)

    norm_table = _normalize_table(word_embeddings, gamma, beta)
    idx = input_ids.reshape(NW, n_chunks, CH).astype(jnp.int32)
    out_flat = _make_sc_gather(ntok, H, n_chunks)(norm_table, idx)
    return out_flat.reshape(B, L, H)


# speedup vs baseline: 1.2085x; 1.2085x over previous
import functools

import jax
import jax.numpy as jnp
from jax import lax
from jax.experimental import pallas as pl
from jax.experimental.pallas import tpu as pltpu
from jax.experimental.pallas import tpu_sc as plsc

EPS = 1e-12

NC = 2
NS = 16
NW = NC * NS

CH = 64
HP = 384
LANES = 16


def _ln_table_body(tbl_ref, g_ref, b_ref, out_ref):
    x = tbl_ref[...]
    u = jnp.mean(x, axis=-1, keepdims=True)
    s = jnp.mean((x - u) ** 2, axis=-1, keepdims=True)
    y = g_ref[...] * ((x - u) / jnp.sqrt(s + EPS)) + b_ref[...]
    out_ref[...] = jnp.pad(y, ((0, 0), (0, HP - y.shape[1])))


def _normalize_table(word_embeddings, gamma, beta):
    V, H = word_embeddings.shape
    BR = 2000
    grid = V // BR
    g2 = gamma.reshape(1, H)
    b2 = beta.reshape(1, H)
    return pl.pallas_call(
        _ln_table_body,
        grid=(grid,),
        in_specs=[
            pl.BlockSpec((BR, H), lambda i: (i, 0)),
            pl.BlockSpec((1, H), lambda i: (0, 0)),
            pl.BlockSpec((1, H), lambda i: (0, 0)),
        ],
        out_specs=pl.BlockSpec((BR, HP), lambda i: (i, 0)),
        out_shape=jax.ShapeDtypeStruct((V, HP), jnp.float32),
    )(word_embeddings, g2, b2)


def _make_sc_gather(ntok, H, n_chunks):
    mesh = plsc.VectorSubcoreMesh(core_axis_name="c", subcore_axis_name="s")
    per_w = n_chunks * CH
    nfull = H // LANES
    tail = H - LANES

    @functools.partial(
        pl.kernel,
        mesh=mesh,
        out_type=jax.ShapeDtypeStruct((ntok, H), jnp.float32),
        scratch_types=[
            pltpu.VMEM((n_chunks, CH), jnp.int32),
            pltpu.VMEM((2, CH, HP), jnp.float32),
            pltpu.VMEM((2, CH, H), jnp.float32),
            pltpu.SemaphoreType.DMA((2,)),
            pltpu.SemaphoreType.DMA((2,)),
        ],
    )
    def gather_kernel(tbl_hbm, idx_hbm, out_hbm, idx_v, r384, r300, gsem, ssem):
        wid = lax.axis_index("s") * NC + lax.axis_index("c")
        base = wid * per_w
        pltpu.sync_copy(idx_hbm.at[wid], idx_v)

        def gather_chunk(c, b):
            return pltpu.make_async_copy(
                tbl_hbm.at[idx_v.at[c]], r384.at[b], gsem.at[b])

        def store_chunk(c, b):
            row0 = pl.multiple_of(base + c * CH, 8)
            return pltpu.make_async_copy(
                r300.at[b], out_hbm.at[pl.ds(row0, CH)], ssem.at[b])

        gather_chunk(0, 0).start()
        gather_chunk(1, 1).start()

        def step(c0):
            for b in range(2):
                c = c0 + b
                gather_chunk(c, b).wait()

                @pl.when(c >= 2)
                def _():
                    store_chunk(c, b).wait()

                def repack(j):
                    vals = [r384[b, j, pl.ds(LANES * k, LANES)]
                            for k in range(nfull)]
                    vals.append(r384[b, j, pl.ds(tail, LANES)])
                    for k in range(nfull):
                        r300[b, j, pl.ds(LANES * k, LANES)] = vals[k]
                    r300[b, j, pl.ds(tail, LANES)] = vals[nfull]

                pl.loop(0, CH, unroll=2)(repack)

                @pl.when(c + 2 < n_chunks)
                def _():
                    gather_chunk(c + 2, b).start()

                store_chunk(c, b).start()

        pl.loop(0, n_chunks, step=2)(step)
        store_chunk(n_chunks - 2, 0).wait()
        store_chunk(n_chunks - 1, 1).wait()

    return gather_kernel


def kernel(input_ids, word_embeddings, gamma, beta):
    B, L = input_ids.shape
    V, H = word_embeddings.shape
    ntok = B * L
    n_chunks = ntok // (NW * CH)

    norm_table = _normalize_table(word_embeddings, gamma, beta)
    idx = input_ids.reshape(NW, n_chunks, CH).astype(jnp.int32)
    out_flat = _make_sc_gather(ntok, H, n_chunks)(norm_table, idx)
    return out_flat.reshape(B, L, H)
